# baseline (speedup 1.0000x reference)
import functools

import jax
import jax.numpy as jnp
from jax import lax
from jax.experimental import pallas as pl
from jax.experimental.pallas import tpu as pltpu
from jax.experimental.pallas import tpu_sc as plsc

VOCAB_BLOCK = 2048
TR_BLOCK = 1792


def _tr_body(lo_ref, hi_ref, o_ref):
    o_ref[:, :64] = lax.transpose(lo_ref[...], (1, 0))
    o_ref[:, 64:] = lax.transpose(hi_ref[...], (1, 0))


def _pack_table(Et, V2P):
    D, V = Et.shape
    BVT = TR_BLOCK
    nh = V2P // BVT
    assert nh * BVT == V2P
    return pl.pallas_call(
        _tr_body,
        grid=(nh,),
        in_specs=[
            pl.BlockSpec((D, BVT), lambda i: (0, i)),
            pl.BlockSpec((D, BVT), lambda i, _n=nh: (0, i + _n)),
        ],
        out_specs=pl.BlockSpec((BVT, 128), lambda i: (i, 0)),
        out_shape=jax.ShapeDtypeStruct((V2P, 128), jnp.float32),
    )(Et, Et)


def _make_sc_gather(V2P: int, B: int):
    info = plsc.get_sparse_core_info()
    NC, NS, L = info.num_cores, info.num_subcores, info.num_lanes
    NW = NC * NS
    assert B % (8 * NW) == 0
    b_per_w = B // NW
    mesh = plsc.VectorSubcoreMesh(core_axis_name="c", subcore_axis_name="s")

    @functools.partial(
        pl.kernel,
        mesh=mesh,
        out_type=jax.ShapeDtypeStruct((B, 128), jnp.float32),
        scratch_types=[
            pltpu.VMEM((b_per_w,), jnp.int32),
            pltpu.VMEM((b_per_w, 128), jnp.float32),
            pltpu.SemaphoreType.DMA,
        ],
        compiler_params=pltpu.CompilerParams(use_tc_tiling_on_sc=True),
    )
    def gather(table_hbm, idx_hbm, out_hbm, idx_v, rows_v, sem):
        wid = lax.axis_index("s") * NC + lax.axis_index("c")
        base = wid * b_per_w
        pltpu.sync_copy(idx_hbm.at[pl.ds(base, b_per_w)], idx_v)
        for c in range(b_per_w // L):
            v = idx_v[pl.ds(L * c, L)]
            idx_v[pl.ds(L * c, L)] = jnp.where(v < V2P, v, v - V2P)
        pltpu.async_copy(table_hbm.at[idx_v], rows_v, sem).wait()
        pltpu.sync_copy(rows_v, out_hbm.at[pl.ds(base, b_per_w)])

    return gather


def _mm_body(et_ref, w_ref, b_ref, o_ref):
    acc = lax.dot_general(
        w_ref[...],
        et_ref[...],
        (((0,), (0,)), ((), ())),
        preferred_element_type=jnp.float32,
    )
    bcol = lax.transpose(b_ref[...], (1, 0))
    o_ref[...] = acc + bcol


def _projection_t(eT, Wt, b2d):
    D, B = eT.shape
    V = Wt.shape[1]
    BV = VOCAB_BLOCK
    grid = (pl.cdiv(V, BV),)
    return pl.pallas_call(
        _mm_body,
        grid=grid,
        in_specs=[
            pl.BlockSpec((D, B), lambda i: (0, 0)),
            pl.BlockSpec((D, BV), lambda i: (0, i)),
            pl.BlockSpec((1, BV), lambda i: (0, i)),
        ],
        out_specs=pl.BlockSpec((BV, B), lambda i: (i, 0)),
        out_shape=jax.ShapeDtypeStruct((V, B), jnp.float32),
        compiler_params=pltpu.CompilerParams(
            fuse_transposed_lhs_in_matmul=True,
        ),
    )(eT, Wt, b2d)


def kernel(inputs, embeddings, W, b):
    V, D = embeddings.shape
    B = inputs.shape[0]
    V2P = ((V // 2 + TR_BLOCK - 1) // TR_BLOCK) * TR_BLOCK
    table_pair = _pack_table(embeddings.T, V2P)
    gather = _make_sc_gather(V2P, B)
    idx = inputs.astype(jnp.int32)
    e128 = gather(table_pair, idx)
    e = jnp.where((idx < V2P)[:, None], e128[:, :D], e128[:, D:])
    out_t = _projection_t(e.T, W.T, b.reshape(1, V))
    return out_t.T


# speedup vs baseline: 1.0152x; 1.0152x over previous
import functools

import jax
import jax.numpy as jnp
from jax import lax
from jax.experimental import pallas as pl
from jax.experimental.pallas import tpu as pltpu
from jax.experimental.pallas import tpu_sc as plsc

VOCAB_BLOCK = 2048
TR_BLOCK = 2176


def _tr_body(lo_ref, hi_ref, o_ref):
    o_ref[:, :64] = lax.transpose(lo_ref[...], (1, 0))
    o_ref[:, 64:] = lax.transpose(hi_ref[...], (1, 0))


def _pack_table(Et, V2P):
    D, V = Et.shape
    BVT = TR_BLOCK
    nh = V2P // BVT
    assert nh * BVT == V2P
    return pl.pallas_call(
        _tr_body,
        grid=(nh,),
        in_specs=[
            pl.BlockSpec((D, BVT), lambda i: (0, i)),
            pl.BlockSpec((D, BVT), lambda i, _n=nh: (0, i + _n)),
        ],
        out_specs=pl.BlockSpec((BVT, 128), lambda i: (i, 0)),
        out_shape=jax.ShapeDtypeStruct((V2P, 128), jnp.float32),
    )(Et, Et)


def _make_sc_gather(V2P: int, B: int):
    info = plsc.get_sparse_core_info()
    NC, NS, L = info.num_cores, info.num_subcores, info.num_lanes
    NW = NC * NS
    assert B % (8 * NW) == 0
    b_per_w = B // NW
    mesh = plsc.VectorSubcoreMesh(core_axis_name="c", subcore_axis_name="s")

    @functools.partial(
        pl.kernel,
        mesh=mesh,
        out_type=jax.ShapeDtypeStruct((B, 128), jnp.float32),
        scratch_types=[
            pltpu.VMEM((b_per_w,), jnp.int32),
            pltpu.VMEM((b_per_w, 128), jnp.float32),
            pltpu.SemaphoreType.DMA,
        ],
        compiler_params=pltpu.CompilerParams(use_tc_tiling_on_sc=True),
    )
    def gather(table_hbm, idx_hbm, out_hbm, idx_v, rows_v, sem):
        wid = lax.axis_index("s") * NC + lax.axis_index("c")
        base = wid * b_per_w
        pltpu.sync_copy(idx_hbm.at[pl.ds(base, b_per_w)], idx_v)
        for c in range(b_per_w // L):
            v = idx_v[pl.ds(L * c, L)]
            idx_v[pl.ds(L * c, L)] = jnp.where(v < V2P, v, v - V2P)
        pltpu.async_copy(table_hbm.at[idx_v], rows_v, sem).wait()
        pltpu.sync_copy(rows_v, out_hbm.at[pl.ds(base, b_per_w)])

    return gather


def _mm_body(et_ref, w_ref, b_ref, o_ref):
    acc = lax.dot_general(
        w_ref[...],
        et_ref[...],
        (((0,), (0,)), ((), ())),
        preferred_element_type=jnp.float32,
    )
    bcol = lax.transpose(b_ref[...], (1, 0))
    o_ref[...] = acc + bcol


def _projection_t(eT, Wt, b2d):
    D, B = eT.shape
    V = Wt.shape[1]
    BV = VOCAB_BLOCK
    grid = (pl.cdiv(V, BV),)
    return pl.pallas_call(
        _mm_body,
        grid=grid,
        in_specs=[
            pl.BlockSpec((D, B), lambda i: (0, 0)),
            pl.BlockSpec((D, BV), lambda i: (0, i)),
            pl.BlockSpec((1, BV), lambda i: (0, i)),
        ],
        out_specs=pl.BlockSpec((BV, B), lambda i: (i, 0)),
        out_shape=jax.ShapeDtypeStruct((V, B), jnp.float32),
        compiler_params=pltpu.CompilerParams(
            fuse_transposed_lhs_in_matmul=True,
        ),
    )(eT, Wt, b2d)


def kernel(inputs, embeddings, W, b):
    V, D = embeddings.shape
    B = inputs.shape[0]
    V2P = ((V // 2 + TR_BLOCK - 1) // TR_BLOCK) * TR_BLOCK
    table_pair = _pack_table(embeddings.T, V2P)
    gather = _make_sc_gather(V2P, B)
    idx = inputs.astype(jnp.int32)
    e128 = gather(table_pair, idx)
    e = jnp.where((idx < V2P)[:, None], e128[:, :D], e128[:, D:])
    out_t = _projection_t(e.T, W.T, b.reshape(1, V))
    return out_t.T
